# baseline (speedup 1.0000x reference)
import jax
import jax.numpy as jnp
from jax.experimental import pallas as pl
from jax.experimental.pallas import tpu as pltpu

_R = 16
_NS = 2 * _R
_STRIDE = 2
_EPS = 1e-5
_OFFS = list(range(-_R, 0)) + list(range(1, _R + 1))


def _slab(even, odd, o, nd):
    if o % 2 == 0:
        base = _R // 2 + o // 2
        return even[base:base + nd, :]
    base = _R // 2 + (o - 1) // 2
    return odd[base:base + nd, :]


def _ext(x, hal):
    n = x.shape[0]
    return jnp.concatenate([x[n - hal:], x, x[:hal]], axis=0)


def _pw(p, w1):
    return jnp.dot(p, w1[0:2, :], preferred_element_type=jnp.float32)


def _fused_body(fe_ref, fo_ref, pe_ref, po_ref, w1_ref, wa_ref,
                g1_ref, b1_ref, g2_ref, b2_ref, w2_ref, out_ref,
                ge_s, go_s, ze_s, zo_s, pwe_s, acc_s, st_s):
    ph = pl.program_id(0)
    b = pl.program_id(1)
    Bv = ge_s.shape[0]
    nd = fe_ref.shape[1]
    C = fe_ref.shape[2]
    hal = _R // 2
    w1 = w1_ref[...]

    @pl.when(ph == 0)
    def _phase0():
        fe = fe_ref[0]
        fo = fo_ref[0]
        pwe_b = _pw(pe_ref[0], w1)
        pwo_b = _pw(po_ref[0], w1)
        w1b = w1[2:, :]
        w2 = w2_ref[...]
        ge = pwe_b + jnp.dot(fe, w1b, preferred_element_type=jnp.float32)
        go = pwo_b + jnp.dot(fo, w1b, preferred_element_type=jnp.float32)
        ze = jnp.dot(fe, w2, preferred_element_type=jnp.float32)
        zo = jnp.dot(fo, w2, preferred_element_type=jnp.float32)
        ge_s[b] = ge
        go_s[b] = go
        ze_s[b] = ze
        zo_s[b] = zo
        pwe_s[b] = pwe_b

        @pl.when(b == 0)
        def _():
            acc_s[...] = jnp.zeros_like(acc_s)

        acc_s[0:1] += jnp.sum(ze, axis=0, keepdims=True) + jnp.sum(zo, axis=0, keepdims=True)
        acc_s[1:2] += jnp.sum(ze * ze, axis=0, keepdims=True) + jnp.sum(zo * zo, axis=0, keepdims=True)
        acc_s[2:3] += jnp.sum(ge, axis=0, keepdims=True) + jnp.sum(go, axis=0, keepdims=True)
        acc_s[3:4] += jnp.sum(ge * ge, axis=0, keepdims=True) + jnp.sum(go * go, axis=0, keepdims=True)

        ge_b = _ext(ge, hal)
        go_b = _ext(go, hal)
        s = _slab(ge_b, go_b, _OFFS[0], nd)
        for o in _OFFS[1:]:
            s = s + _slab(ge_b, go_b, o, nd)
        acc_s[4:5] += jnp.sum(pwe_b * s, axis=0, keepdims=True)
        acc_s[5:6] += jnp.sum(pwe_b, axis=0, keepdims=True)
        acc_s[6:7] += jnp.sum(pwe_b * pwe_b, axis=0, keepdims=True)

    @pl.when(ph == 1)
    def _phase1():

        @pl.when(b == 0)
        def _():
            nrows = Bv * nd * 2
            mu2 = acc_s[0:1] / nrows
            var2 = acc_s[1:2] / nrows - mu2 * mu2
            s2 = g2_ref[...] / jnp.sqrt(var2 + _EPS)
            t2 = b2_ref[...] - mu2 * s2
            cnt = Bv * nd * _NS
            mult = _NS // _STRIDE
            sum_h = mult * acc_s[2:3] - _NS * acc_s[5:6]
            ssq_h = mult * acc_s[3:4] - 2.0 * acc_s[4:5] + _NS * acc_s[6:7]
            mu1 = sum_h / cnt
            var1 = ssq_h / cnt - mu1 * mu1
            s1 = g1_ref[...] / jnp.sqrt(var1 + _EPS)
            t1 = b1_ref[...] - mu1 * s1
            st_s[0:1] = s1
            st_s[1:2] = t1
            st_s[2:3] = s2
            st_s[3:4] = t2

        s1 = st_s[0:1]
        t1 = st_s[1:2]
        s2 = st_s[2:3]
        t2 = st_s[3:4]
        wa = wa_ref[...]

        qd = t1 - pwe_s[b] * s1
        gse = _ext(ge_s[b] * s1, hal)
        gso = _ext(go_s[b] * s1, hal)
        yne = _ext(jnp.maximum(ze_s[b] * s2 + t2, 0.0), hal)
        yno = _ext(jnp.maximum(zo_s[b] * s2 + t2, 0.0), hal)

        logits = []
        for o in _OFFS:
            a = jnp.maximum(_slab(gse, gso, o, nd) + qd, 0.0)
            logits.append(jnp.dot(a, wa, preferred_element_type=jnp.float32))
        lg = jnp.concatenate(logits, axis=1)
        lg = lg - jnp.max(lg, axis=1, keepdims=True)
        e = jnp.exp(lg)
        w = e / jnp.sum(e, axis=1, keepdims=True)

        acc = w[:, 0:1] * _slab(yne, yno, _OFFS[0], nd)
        for j, o in enumerate(_OFFS[1:]):
            acc += w[:, j + 1:j + 2] * _slab(yne, yno, o, nd)
        out_ref[0] = acc


def kernel(points, features, W1, g1, b1, Wa, ba, W2, g2, b2):
    Bv, Nv, _ = points.shape
    C = features.shape[1]
    nd = Nv // _STRIDE

    f3 = features.reshape(Bv, Nv, C)
    fe = f3[:, 0::2]
    fo = f3[:, 1::2]
    pe = points[:, 0::2]
    po = points[:, 1::2]

    bmap = lambda ph, b: (b, 0, 0)
    pmap = lambda ph, b: (b, 0, 0)
    cmap2 = lambda ph, b: (0, 0)

    out = pl.pallas_call(
        _fused_body,
        grid=(2, Bv),
        in_specs=[
            pl.BlockSpec((1, nd, C), bmap),
            pl.BlockSpec((1, nd, C), bmap),
            pl.BlockSpec((1, nd, 2), pmap),
            pl.BlockSpec((1, nd, 2), bmap),
            pl.BlockSpec(W1.shape, cmap2),
            pl.BlockSpec(Wa.shape, cmap2),
            pl.BlockSpec((1, C), cmap2),
            pl.BlockSpec((1, C), cmap2),
            pl.BlockSpec((1, C), cmap2),
            pl.BlockSpec((1, C), cmap2),
            pl.BlockSpec(W2.shape, cmap2),
        ],
        out_specs=pl.BlockSpec((1, nd, C), pmap),
        out_shape=jax.ShapeDtypeStruct((Bv, nd, C), jnp.float32),
        scratch_shapes=[
            pltpu.VMEM((Bv, nd, C), jnp.float32),
            pltpu.VMEM((Bv, nd, C), jnp.float32),
            pltpu.VMEM((Bv, nd, C), jnp.float32),
            pltpu.VMEM((Bv, nd, C), jnp.float32),
            pltpu.VMEM((Bv, nd, C), jnp.float32),
            pltpu.VMEM((7, C), jnp.float32),
            pltpu.VMEM((4, C), jnp.float32),
        ],
    )(fe, fo, pe, po, W1, Wa,
      g1.reshape(1, C), b1.reshape(1, C), g2.reshape(1, C), b2.reshape(1, C),
      W2)

    return (pe, out.reshape(Bv * nd, C))


# speedup vs baseline: 1.0110x; 1.0110x over previous
import jax
import jax.numpy as jnp
from jax.experimental import pallas as pl
from jax.experimental.pallas import tpu as pltpu

_R = 16
_NS = 2 * _R
_STRIDE = 2
_EPS = 1e-5
_OFFS = list(range(-_R, 0)) + list(range(1, _R + 1))


def _slab(even, odd, o, nd):
    if o % 2 == 0:
        base = _R // 2 + o // 2
        return even[base:base + nd, :]
    base = _R // 2 + (o - 1) // 2
    return odd[base:base + nd, :]


def _ext(x, hal):
    n = x.shape[0]
    return jnp.concatenate([x[n - hal:], x, x[:hal]], axis=0)


def _pw(p, w1):
    return p[:, 0:1] * w1[0:1, :] + p[:, 1:2] * w1[1:2, :]


def _fused_body(fe_ref, fo_ref, pe_ref, po_ref, w1_ref, wa_ref,
                g1_ref, b1_ref, g2_ref, b2_ref, w2_ref, out_ref,
                ge_s, go_s, ze_s, zo_s, pwe_s, acc_s, st_s):
    ph = pl.program_id(0)
    b = pl.program_id(1)
    Bv = ge_s.shape[0]
    nd = fe_ref.shape[1]
    C = fe_ref.shape[2]
    hal = _R // 2
    w1 = w1_ref[...]

    @pl.when(ph == 0)
    def _phase0():
        fe = fe_ref[0]
        fo = fo_ref[0]
        pwe_b = _pw(pe_ref[0], w1)
        pwo_b = _pw(po_ref[0], w1)
        w1b = w1[2:, :]
        w2 = w2_ref[...]
        ge = pwe_b + jnp.dot(fe, w1b, preferred_element_type=jnp.float32)
        go = pwo_b + jnp.dot(fo, w1b, preferred_element_type=jnp.float32)
        ze = jnp.dot(fe, w2, preferred_element_type=jnp.float32)
        zo = jnp.dot(fo, w2, preferred_element_type=jnp.float32)
        ge_s[b] = ge
        go_s[b] = go
        ze_s[b] = ze
        zo_s[b] = zo
        pwe_s[b] = pwe_b

        @pl.when(b == 0)
        def _():
            acc_s[...] = jnp.zeros_like(acc_s)

        acc_s[0:1] += jnp.sum(ze, axis=0, keepdims=True) + jnp.sum(zo, axis=0, keepdims=True)
        acc_s[1:2] += jnp.sum(ze * ze, axis=0, keepdims=True) + jnp.sum(zo * zo, axis=0, keepdims=True)
        acc_s[2:3] += jnp.sum(ge, axis=0, keepdims=True) + jnp.sum(go, axis=0, keepdims=True)
        acc_s[3:4] += jnp.sum(ge * ge, axis=0, keepdims=True) + jnp.sum(go * go, axis=0, keepdims=True)

        ge_b = _ext(ge, hal)
        go_b = _ext(go, hal)
        s = _slab(ge_b, go_b, _OFFS[0], nd)
        for o in _OFFS[1:]:
            s = s + _slab(ge_b, go_b, o, nd)
        acc_s[4:5] += jnp.sum(pwe_b * s, axis=0, keepdims=True)
        acc_s[5:6] += jnp.sum(pwe_b, axis=0, keepdims=True)
        acc_s[6:7] += jnp.sum(pwe_b * pwe_b, axis=0, keepdims=True)

    @pl.when(ph == 1)
    def _phase1():

        @pl.when(b == 0)
        def _():
            nrows = Bv * nd * 2
            mu2 = acc_s[0:1] / nrows
            var2 = acc_s[1:2] / nrows - mu2 * mu2
            s2 = g2_ref[...] / jnp.sqrt(var2 + _EPS)
            t2 = b2_ref[...] - mu2 * s2
            cnt = Bv * nd * _NS
            mult = _NS // _STRIDE
            sum_h = mult * acc_s[2:3] - _NS * acc_s[5:6]
            ssq_h = mult * acc_s[3:4] - 2.0 * acc_s[4:5] + _NS * acc_s[6:7]
            mu1 = sum_h / cnt
            var1 = ssq_h / cnt - mu1 * mu1
            s1 = g1_ref[...] / jnp.sqrt(var1 + _EPS)
            t1 = b1_ref[...] - mu1 * s1
            st_s[0:1] = s1
            st_s[1:2] = t1
            st_s[2:3] = s2
            st_s[3:4] = t2

        s1 = st_s[0:1]
        t1 = st_s[1:2]
        s2 = st_s[2:3]
        t2 = st_s[3:4]
        wa = wa_ref[...]

        qd = t1 - pwe_s[b] * s1
        gse = _ext(ge_s[b] * s1, hal)
        gso = _ext(go_s[b] * s1, hal)
        yne = _ext(jnp.maximum(ze_s[b] * s2 + t2, 0.0), hal)
        yno = _ext(jnp.maximum(zo_s[b] * s2 + t2, 0.0), hal)

        logits = []
        for o in _OFFS:
            a = jnp.maximum(_slab(gse, gso, o, nd) + qd, 0.0)
            logits.append(jnp.dot(a, wa, preferred_element_type=jnp.float32))
        lg = jnp.concatenate(logits, axis=1)
        lg = lg - jnp.max(lg, axis=1, keepdims=True)
        e = jnp.exp(lg)
        w = e / jnp.sum(e, axis=1, keepdims=True)

        acc = w[:, 0:1] * _slab(yne, yno, _OFFS[0], nd)
        for j, o in enumerate(_OFFS[1:]):
            acc += w[:, j + 1:j + 2] * _slab(yne, yno, o, nd)
        out_ref[0] = acc


def kernel(points, features, W1, g1, b1, Wa, ba, W2, g2, b2):
    Bv, Nv, _ = points.shape
    C = features.shape[1]
    nd = Nv // _STRIDE

    f3 = features.reshape(Bv, Nv, C)
    fe = f3[:, 0::2]
    fo = f3[:, 1::2]
    pe = points[:, 0::2]
    po = points[:, 1::2]

    bmap = lambda ph, b: (b, 0, 0)
    pmap = lambda ph, b: (b, 0, 0)
    cmap2 = lambda ph, b: (0, 0)

    out = pl.pallas_call(
        _fused_body,
        grid=(2, Bv),
        in_specs=[
            pl.BlockSpec((1, nd, C), bmap),
            pl.BlockSpec((1, nd, C), bmap),
            pl.BlockSpec((1, nd, 2), pmap),
            pl.BlockSpec((1, nd, 2), bmap),
            pl.BlockSpec(W1.shape, cmap2),
            pl.BlockSpec(Wa.shape, cmap2),
            pl.BlockSpec((1, C), cmap2),
            pl.BlockSpec((1, C), cmap2),
            pl.BlockSpec((1, C), cmap2),
            pl.BlockSpec((1, C), cmap2),
            pl.BlockSpec(W2.shape, cmap2),
        ],
        out_specs=pl.BlockSpec((1, nd, C), pmap),
        out_shape=jax.ShapeDtypeStruct((Bv, nd, C), jnp.float32),
        scratch_shapes=[
            pltpu.VMEM((Bv, nd, C), jnp.float32),
            pltpu.VMEM((Bv, nd, C), jnp.float32),
            pltpu.VMEM((Bv, nd, C), jnp.float32),
            pltpu.VMEM((Bv, nd, C), jnp.float32),
            pltpu.VMEM((Bv, nd, C), jnp.float32),
            pltpu.VMEM((7, C), jnp.float32),
            pltpu.VMEM((4, C), jnp.float32),
        ],
    )(fe, fo, pe, po, W1, Wa,
      g1.reshape(1, C), b1.reshape(1, C), g2.reshape(1, C), b2.reshape(1, C),
      W2)

    return (pe, out.reshape(Bv * nd, C))


# speedup vs baseline: 1.1027x; 1.0907x over previous
import jax
import jax.numpy as jnp
from jax.experimental import pallas as pl
from jax.experimental.pallas import tpu as pltpu

_R = 16
_NS = 2 * _R
_STRIDE = 2
_EPS = 1e-5
_OFFS = list(range(-_R, 0)) + list(range(1, _R + 1))


def _slab(even, odd, o, nd):
    if o % 2 == 0:
        base = _R // 2 + o // 2
        return even[base:base + nd, :]
    base = _R // 2 + (o - 1) // 2
    return odd[base:base + nd, :]


def _ext(x, hal):
    n = x.shape[0]
    return jnp.concatenate([x[n - hal:], x, x[:hal]], axis=0)


def _pw(p, w1):
    return p[:, 0:1] * w1[0:1, :] + p[:, 1:2] * w1[1:2, :]


def _fused_body(fe_ref, fo_ref, pe_ref, po_ref, w1_ref, wa_ref,
                g1_ref, b1_ref, g2_ref, b2_ref, w2_ref, out_ref,
                ge_s, go_s, ze_s, zo_s, acc_s, st_s):
    ph = pl.program_id(0)
    b = pl.program_id(1)
    Bv = ge_s.shape[0]
    nd = fe_ref.shape[1]
    C = fe_ref.shape[2]
    hal = _R // 2
    w1 = w1_ref[...]

    @pl.when(ph == 0)
    def _phase0():
        fe = fe_ref[0]
        fo = fo_ref[0]
        pwe_b = _pw(pe_ref[0], w1)
        pwo_b = _pw(po_ref[0], w1)
        w1b = w1[2:, :]
        w2 = w2_ref[...]
        ge = pwe_b + jnp.dot(fe, w1b, preferred_element_type=jnp.float32)
        go = pwo_b + jnp.dot(fo, w1b, preferred_element_type=jnp.float32)
        ze = jnp.dot(fe, w2, preferred_element_type=jnp.float32)
        zo = jnp.dot(fo, w2, preferred_element_type=jnp.float32)
        ge_s[b] = ge
        go_s[b] = go
        ze_s[b] = ze
        zo_s[b] = zo

        @pl.when(b == 0)
        def _():
            acc_s[...] = jnp.zeros_like(acc_s)

        acc_s[0:1] += jnp.sum(ze, axis=0, keepdims=True) + jnp.sum(zo, axis=0, keepdims=True)
        acc_s[1:2] += jnp.sum(ze * ze, axis=0, keepdims=True) + jnp.sum(zo * zo, axis=0, keepdims=True)
        acc_s[2:3] += jnp.sum(ge, axis=0, keepdims=True) + jnp.sum(go, axis=0, keepdims=True)
        acc_s[3:4] += jnp.sum(ge * ge, axis=0, keepdims=True) + jnp.sum(go * go, axis=0, keepdims=True)

        ge_b = _ext(ge, hal)
        go_b = _ext(go, hal)
        s = _slab(ge_b, go_b, _OFFS[0], nd)
        for o in _OFFS[1:]:
            s = s + _slab(ge_b, go_b, o, nd)
        acc_s[4:5] += jnp.sum(pwe_b * s, axis=0, keepdims=True)
        acc_s[5:6] += jnp.sum(pwe_b, axis=0, keepdims=True)
        acc_s[6:7] += jnp.sum(pwe_b * pwe_b, axis=0, keepdims=True)

    @pl.when(ph == 1)
    def _phase1():

        @pl.when(b == 0)
        def _():
            nrows = Bv * nd * 2
            mu2 = acc_s[0:1] / nrows
            var2 = acc_s[1:2] / nrows - mu2 * mu2
            s2 = g2_ref[...] / jnp.sqrt(var2 + _EPS)
            t2 = b2_ref[...] - mu2 * s2
            cnt = Bv * nd * _NS
            mult = _NS // _STRIDE
            sum_h = mult * acc_s[2:3] - _NS * acc_s[5:6]
            ssq_h = mult * acc_s[3:4] - 2.0 * acc_s[4:5] + _NS * acc_s[6:7]
            mu1 = sum_h / cnt
            var1 = ssq_h / cnt - mu1 * mu1
            s1 = g1_ref[...] / jnp.sqrt(var1 + _EPS)
            t1 = b1_ref[...] - mu1 * s1
            st_s[0:1] = s1
            st_s[1:2] = t1
            st_s[2:3] = s2
            st_s[3:4] = t2

        s1 = st_s[0:1]
        t1 = st_s[1:2]
        s2 = st_s[2:3]
        t2 = st_s[3:4]
        wa = wa_ref[...]

        qd = t1 - _pw(pe_ref[0], w1) * s1
        gse = _ext(ge_s[b] * s1, hal)
        gso = _ext(go_s[b] * s1, hal)
        yne = _ext(jnp.maximum(ze_s[b] * s2 + t2, 0.0), hal)
        yno = _ext(jnp.maximum(zo_s[b] * s2 + t2, 0.0), hal)

        logits = []
        for o in _OFFS:
            a = jnp.maximum(_slab(gse, gso, o, nd) + qd, 0.0)
            logits.append(jnp.dot(a, wa, preferred_element_type=jnp.float32))
        lg = jnp.concatenate(logits, axis=1)
        lg = lg - jnp.max(lg, axis=1, keepdims=True)
        e = jnp.exp(lg)
        w = e / jnp.sum(e, axis=1, keepdims=True)

        acc = w[:, 0:1] * _slab(yne, yno, _OFFS[0], nd)
        for j, o in enumerate(_OFFS[1:]):
            acc += w[:, j + 1:j + 2] * _slab(yne, yno, o, nd)
        out_ref[0] = acc


def kernel(points, features, W1, g1, b1, Wa, ba, W2, g2, b2):
    Bv, Nv, _ = points.shape
    C = features.shape[1]
    nd = Nv // _STRIDE

    f3 = features.reshape(Bv, Nv, C)
    fe = f3[:, 0::2]
    fo = f3[:, 1::2]
    pe = points[:, 0::2]
    po = points[:, 1::2]

    bmap = lambda ph, b: (b, 0, 0)
    pmap = lambda ph, b: (b, 0, 0)
    cmap2 = lambda ph, b: (0, 0)

    out = pl.pallas_call(
        _fused_body,
        grid=(2, Bv),
        in_specs=[
            pl.BlockSpec((1, nd, C), bmap),
            pl.BlockSpec((1, nd, C), bmap),
            pl.BlockSpec((1, nd, 2), pmap),
            pl.BlockSpec((1, nd, 2), bmap),
            pl.BlockSpec(W1.shape, cmap2),
            pl.BlockSpec(Wa.shape, cmap2),
            pl.BlockSpec((1, C), cmap2),
            pl.BlockSpec((1, C), cmap2),
            pl.BlockSpec((1, C), cmap2),
            pl.BlockSpec((1, C), cmap2),
            pl.BlockSpec(W2.shape, cmap2),
        ],
        out_specs=pl.BlockSpec((1, nd, C), pmap),
        out_shape=jax.ShapeDtypeStruct((Bv, nd, C), jnp.float32),
        scratch_shapes=[
            pltpu.VMEM((Bv, nd, C), jnp.float32),
            pltpu.VMEM((Bv, nd, C), jnp.float32),
            pltpu.VMEM((Bv, nd, C), jnp.float32),
            pltpu.VMEM((Bv, nd, C), jnp.float32),
            pltpu.VMEM((7, C), jnp.float32),
            pltpu.VMEM((4, C), jnp.float32),
        ],
    )(fe, fo, pe, po, W1, Wa,
      g1.reshape(1, C), b1.reshape(1, C), g2.reshape(1, C), b2.reshape(1, C),
      W2)

    return (pe, out.reshape(Bv * nd, C))


# speedup vs baseline: 1.1027x; 1.0000x over previous
import jax
import jax.numpy as jnp
from jax.experimental import pallas as pl
from jax.experimental.pallas import tpu as pltpu

_R = 16
_NS = 2 * _R
_STRIDE = 2
_EPS = 1e-5
_OFFS = list(range(-_R, 0)) + list(range(1, _R + 1))


def _slab(even, odd, o, nd):
    if o % 2 == 0:
        base = _R // 2 + o // 2
        return even[base:base + nd, :]
    base = _R // 2 + (o - 1) // 2
    return odd[base:base + nd, :]


def _shifts(x, nd):
    return [x[r:r + nd + 2 * (_R // 2) - r, :] for r in range(8)]


def _slab8(even_sh, odd_sh, o, nd):
    if o % 2 == 0:
        base = _R // 2 + o // 2
    else:
        base = _R // 2 + (o - 1) // 2
    q, r = divmod(base, 8)
    sh = even_sh if o % 2 == 0 else odd_sh
    return sh[r][q * 8:q * 8 + nd, :]


def _ext(x, hal):
    n = x.shape[0]
    return jnp.concatenate([x[n - hal:], x, x[:hal]], axis=0)


def _pw(p, w1):
    return p[:, 0:1] * w1[0:1, :] + p[:, 1:2] * w1[1:2, :]


def _fused_body(fe_ref, fo_ref, pe_ref, po_ref, w1_ref, wa_ref,
                g1_ref, b1_ref, g2_ref, b2_ref, w2_ref, out_ref,
                ge_s, go_s, ze_s, zo_s, acc_s, st_s):
    ph = pl.program_id(0)
    b = pl.program_id(1)
    Bv = ge_s.shape[0]
    nd = fe_ref.shape[1]
    C = fe_ref.shape[2]
    hal = _R // 2
    w1 = w1_ref[...]

    @pl.when(ph == 0)
    def _phase0():
        fe = fe_ref[0]
        fo = fo_ref[0]
        pwe_b = _pw(pe_ref[0], w1)
        pwo_b = _pw(po_ref[0], w1)
        w1b = w1[2:, :]
        w2 = w2_ref[...]
        ge = pwe_b + jnp.dot(fe, w1b, preferred_element_type=jnp.float32)
        go = pwo_b + jnp.dot(fo, w1b, preferred_element_type=jnp.float32)
        ze = jnp.dot(fe, w2, preferred_element_type=jnp.float32)
        zo = jnp.dot(fo, w2, preferred_element_type=jnp.float32)
        ge_s[b] = ge
        go_s[b] = go
        ze_s[b] = ze
        zo_s[b] = zo

        @pl.when(b == 0)
        def _():
            acc_s[...] = jnp.zeros_like(acc_s)

        acc_s[0:1] += jnp.sum(ze, axis=0, keepdims=True) + jnp.sum(zo, axis=0, keepdims=True)
        acc_s[1:2] += jnp.sum(ze * ze, axis=0, keepdims=True) + jnp.sum(zo * zo, axis=0, keepdims=True)
        acc_s[2:3] += jnp.sum(ge, axis=0, keepdims=True) + jnp.sum(go, axis=0, keepdims=True)
        acc_s[3:4] += jnp.sum(ge * ge, axis=0, keepdims=True) + jnp.sum(go * go, axis=0, keepdims=True)

        ge_b = _ext(ge, hal)
        go_b = _ext(go, hal)
        s = _slab(ge_b, go_b, _OFFS[0], nd)
        for o in _OFFS[1:]:
            s = s + _slab(ge_b, go_b, o, nd)
        acc_s[4:5] += jnp.sum(pwe_b * s, axis=0, keepdims=True)
        acc_s[5:6] += jnp.sum(pwe_b, axis=0, keepdims=True)
        acc_s[6:7] += jnp.sum(pwe_b * pwe_b, axis=0, keepdims=True)

    @pl.when(ph == 1)
    def _phase1():

        @pl.when(b == 0)
        def _():
            nrows = Bv * nd * 2
            mu2 = acc_s[0:1] / nrows
            var2 = acc_s[1:2] / nrows - mu2 * mu2
            s2 = g2_ref[...] / jnp.sqrt(var2 + _EPS)
            t2 = b2_ref[...] - mu2 * s2
            cnt = Bv * nd * _NS
            mult = _NS // _STRIDE
            sum_h = mult * acc_s[2:3] - _NS * acc_s[5:6]
            ssq_h = mult * acc_s[3:4] - 2.0 * acc_s[4:5] + _NS * acc_s[6:7]
            mu1 = sum_h / cnt
            var1 = ssq_h / cnt - mu1 * mu1
            s1 = g1_ref[...] / jnp.sqrt(var1 + _EPS)
            t1 = b1_ref[...] - mu1 * s1
            st_s[0:1] = s1
            st_s[1:2] = t1
            st_s[2:3] = s2
            st_s[3:4] = t2

        s1 = st_s[0:1]
        t1 = st_s[1:2]
        s2 = st_s[2:3]
        t2 = st_s[3:4]
        wa = wa_ref[...]

        qd = t1 - _pw(pe_ref[0], w1) * s1
        gse = _shifts(_ext(ge_s[b] * s1, hal), nd)
        gso = _shifts(_ext(go_s[b] * s1, hal), nd)
        yne = _shifts(_ext(jnp.maximum(ze_s[b] * s2 + t2, 0.0), hal), nd)
        yno = _shifts(_ext(jnp.maximum(zo_s[b] * s2 + t2, 0.0), hal), nd)

        logits = []
        for o in _OFFS:
            a = jnp.maximum(_slab8(gse, gso, o, nd) + qd, 0.0)
            logits.append(jnp.dot(a, wa, preferred_element_type=jnp.float32))
        lg = jnp.concatenate(logits, axis=1)
        lg = lg - jnp.max(lg, axis=1, keepdims=True)
        e = jnp.exp(lg)
        w = e / jnp.sum(e, axis=1, keepdims=True)

        acc = w[:, 0:1] * _slab8(yne, yno, _OFFS[0], nd)
        for j, o in enumerate(_OFFS[1:]):
            acc += w[:, j + 1:j + 2] * _slab8(yne, yno, o, nd)
        out_ref[0] = acc


def kernel(points, features, W1, g1, b1, Wa, ba, W2, g2, b2):
    Bv, Nv, _ = points.shape
    C = features.shape[1]
    nd = Nv // _STRIDE

    f3 = features.reshape(Bv, Nv, C)
    fe = f3[:, 0::2]
    fo = f3[:, 1::2]
    pe = points[:, 0::2]
    po = points[:, 1::2]

    bmap = lambda ph, b: (b, 0, 0)
    pmap = lambda ph, b: (b, 0, 0)
    cmap2 = lambda ph, b: (0, 0)

    out = pl.pallas_call(
        _fused_body,
        grid=(2, Bv),
        in_specs=[
            pl.BlockSpec((1, nd, C), bmap),
            pl.BlockSpec((1, nd, C), bmap),
            pl.BlockSpec((1, nd, 2), pmap),
            pl.BlockSpec((1, nd, 2), bmap),
            pl.BlockSpec(W1.shape, cmap2),
            pl.BlockSpec(Wa.shape, cmap2),
            pl.BlockSpec((1, C), cmap2),
            pl.BlockSpec((1, C), cmap2),
            pl.BlockSpec((1, C), cmap2),
            pl.BlockSpec((1, C), cmap2),
            pl.BlockSpec(W2.shape, cmap2),
        ],
        out_specs=pl.BlockSpec((1, nd, C), pmap),
        out_shape=jax.ShapeDtypeStruct((Bv, nd, C), jnp.float32),
        scratch_shapes=[
            pltpu.VMEM((Bv, nd, C), jnp.float32),
            pltpu.VMEM((Bv, nd, C), jnp.float32),
            pltpu.VMEM((Bv, nd, C), jnp.float32),
            pltpu.VMEM((Bv, nd, C), jnp.float32),
            pltpu.VMEM((7, C), jnp.float32),
            pltpu.VMEM((4, C), jnp.float32),
        ],
    )(fe, fo, pe, po, W1, Wa,
      g1.reshape(1, C), b1.reshape(1, C), g2.reshape(1, C), b2.reshape(1, C),
      W2)

    return (pe, out.reshape(Bv * nd, C))
